# baseline (speedup 1.0000x reference)
import functools

import jax
import jax.numpy as jnp
from jax import lax
from jax.experimental import pallas as pl
from jax.experimental.pallas import tpu as pltpu
from jax.experimental.pallas import tpu_sc as plsc

DIM = 64
NC, NS = 2, 16
NW = NC * NS
GRP = 16
WPAIRS = 256


def _sc_transpose(table_t):
    vocab = table_t.shape[1]
    npairs = vocab // 2
    ppw = npairs // NW
    nfull = ppw // WPAIRS
    tail = ppw - nfull * WPAIRS
    mesh = plsc.VectorSubcoreMesh(core_axis_name="c", subcore_axis_name="s")

    @functools.partial(
        pl.kernel,
        mesh=mesh,
        compiler_params=pltpu.CompilerParams(needs_layout_passes=False),
        out_type=jax.ShapeDtypeStruct((npairs, 2 * DIM), jnp.float32),
        scratch_types=[
            pltpu.VMEM((DIM, 2 * WPAIRS), jnp.float32),
            pltpu.VMEM((DIM, 2 * WPAIRS), jnp.float32),
            pltpu.VMEM((WPAIRS, 2 * DIM), jnp.float32),
            pltpu.SemaphoreType.DMA,
            pltpu.SemaphoreType.DMA,
        ],
    )
    def k(tt_hbm, dense_hbm, slab_a, slab_b, outbuf, sem_a, sem_b):
        wid = lax.axis_index("s") * NC + lax.axis_index("c")

        def compute(slab, n):
            @pl.loop(0, n)
            def _(p):
                lane = lax.iota(jnp.int32, GRP)
                rr = jnp.full((GRP,), 0, jnp.int32) + 2 * p
                for j in range(8):
                    csel = jnp.full((GRP,), 16 * (j % 4), jnp.int32) + lane
                    rsel = rr + (j // 4)
                    vals = plsc.load_gather(slab, [csel, rsel])
                    outbuf[p, pl.ds(16 * (j % 4) + 64 * (j // 4), GRP)] = vals

        nwin = (vocab // 2) // WPAIRS
        niter = nwin // NW + 1
        for i in range(niter):
            win = wid + i * NW

            @pl.when(win < nwin)
            def _():
                pltpu.sync_copy(
                    tt_hbm.at[:, pl.ds(win * 2 * WPAIRS, 2 * WPAIRS)],
                    slab_a)
                compute(slab_a, WPAIRS)
                pltpu.sync_copy(outbuf,
                                dense_hbm.at[pl.ds(win * WPAIRS, WPAIRS), :])

    return k(table_t)


def _sc_cosine(table2, tail_t, center_idx, context_idx):
    batch = center_idx.shape[0]
    bpw = batch // NW
    chunk = 256
    nchunks = bpw // chunk
    mesh = plsc.VectorSubcoreMesh(core_axis_name="c", subcore_axis_name="s")

    @functools.partial(
        pl.kernel,
        mesh=mesh,
        compiler_params=pltpu.CompilerParams(needs_layout_passes=False),
        out_type=jax.ShapeDtypeStruct((batch,), jnp.float32),
        scratch_types=[
            pltpu.VMEM((bpw,), jnp.int32),
            pltpu.VMEM((bpw,), jnp.int32),
            pltpu.VMEM((bpw,), jnp.int32),
            pltpu.VMEM((bpw,), jnp.int32),
            pltpu.VMEM((chunk, 2 * DIM), jnp.float32),
            pltpu.VMEM((chunk, 2 * DIM), jnp.float32),
            pltpu.VMEM((bpw,), jnp.float32),
            pltpu.VMEM((DIM, 64), jnp.float32),
            pltpu.SemaphoreType.DMA,
            pltpu.SemaphoreType.DMA,
        ],
    )
    def k(table_hbm, tail_hbm, cen_hbm, ctx_hbm, out_hbm,
          rcen_v, rctx_v, pcen_v, pctx_v, dstc_v, dstx_v, out_v, tail_v,
          sem_c, sem_x):
        wid = lax.axis_index("s") * NC + lax.axis_index("c")
        base = wid * bpw
        pltpu.sync_copy(cen_hbm.at[pl.ds(base, bpw)], rcen_v)
        pltpu.sync_copy(ctx_hbm.at[pl.ds(base, bpw)], rctx_v)
        pltpu.sync_copy(tail_hbm, tail_v)

        @pl.loop(0, bpw // GRP)
        def _pairs(g):
            sl = pl.ds(g * GRP, GRP)
            pcen_v[sl] = lax.shift_right_logical(rcen_v[sl], 1)
            pctx_v[sl] = lax.shift_right_logical(rctx_v[sl], 1)

        for c in range(nchunks):
            cbase = c * chunk
            cp_c = pltpu.async_copy(
                table_hbm.at[pcen_v.at[pl.ds(cbase, chunk)]], dstc_v, sem_c)
            cp_x = pltpu.async_copy(
                table_hbm.at[pctx_v.at[pl.ds(cbase, chunk)]], dstx_v, sem_x)
            cp_c.wait()
            cp_x.wait()

            @pl.loop(0, chunk // GRP)
            def _compute(g):
                lane = lax.iota(jnp.int32, GRP)
                rows = g * GRP + lane
                sl = pl.ds(cbase + g * GRP, GRP)
                rc = rcen_v[sl]
                rx = rctx_v[sl]
                pc = (rc & 1) * DIM
                px = (rx & 1) * DIM
                mc = rc >= TAIL_START
                mx = rx >= TAIL_START
                tc_r = jnp.minimum(jnp.maximum(rc - TAIL_START, 0), 63)
                tx_r = jnp.minimum(jnp.maximum(rx - TAIL_START, 0), 63)
                dot = jnp.zeros((GRP,), jnp.float32)
                cc = jnp.zeros((GRP,), jnp.float32)
                xx = jnp.zeros((GRP,), jnp.float32)
                for d in range(DIM):
                    rot = (jnp.full((GRP,), d, jnp.int32) + lane) & (DIM - 1)
                    cv = plsc.load_gather(dstc_v, [rows, pc + rot])
                    xv = plsc.load_gather(dstx_v, [rows, px + rot])
                    cv = jnp.where(mc, plsc.load_gather(tail_v, [rot, tc_r]), cv)
                    xv = jnp.where(mx, plsc.load_gather(tail_v, [rot, tx_r]), xv)
                    dot = dot + cv * xv
                    cc = cc + cv * cv
                    xx = xx + xv * xv
                y = cc * xx
                iy = plsc.bitcast(y, jnp.int32)
                iz = jnp.int32(0x5F3759DF) - lax.shift_right_logical(iy, 1)
                z = plsc.bitcast(iz, jnp.float32)
                for _ in range(3):
                    z = z * (1.5 - 0.5 * y * z * z)
                denom = y * z
                out_v[pl.ds(cbase + g * GRP, GRP)] = dot / (denom + 1e-8)

        pltpu.sync_copy(out_v, out_hbm.at[pl.ds(base, bpw)])

    return k(table2, tail_t, center_idx, context_idx)


TAIL_START = 999936


@jax.jit
def kernel(center_idx, context_idx, table):
    dense = _sc_transpose(table.T)
    tail_t = table[TAIL_START:, :].T
    return _sc_cosine(dense, tail_t,
                      center_idx.astype(jnp.int32),
                      context_idx.astype(jnp.int32))
